# baseline (device time: 38735 ns/iter reference)
import jax
import jax.numpy as jnp
from jax import lax
from jax.experimental import pallas as pl
from jax.experimental.pallas import tpu as pltpu

N_DEV = 4
_GELU_C = 0.7978845608028654


def _gelu(y):
    return 0.5 * y * (1.0 + jnp.tanh(_GELU_C * (y + 0.044715 * y * y * y)))


def kernel(x, w_mat):
    m_per, k = x.shape
    _, n = w_mat.shape
    n_per = n // N_DEV

    def body(x_ref, w_ref, out_ref, y_ref, send_sems, recv_sems):
        my_pos = lax.axis_index("i")

        barrier_sem = pltpu.get_barrier_semaphore()
        for d in range(1, N_DEV):
            pl.semaphore_signal(
                barrier_sem,
                inc=1,
                device_id=((my_pos + d) % N_DEV,),
                device_id_type=pl.DeviceIdType.MESH,
            )
        pl.semaphore_wait(barrier_sem, N_DEV - 1)

        xb = x_ref[:, :].astype(jnp.bfloat16)

        rdmas = []
        for d in (2, 1, 3, 0):
            tgt = (my_pos + d) % N_DEV
            wblk = w_ref[:, pl.ds(tgt * n_per, n_per)].astype(jnp.bfloat16)
            y_blk = _gelu(jnp.dot(xb, wblk, preferred_element_type=jnp.float32))
            if d == 0:
                out_ref[pl.ds(my_pos * m_per, m_per), :] = y_blk
            else:
                y_ref[:, pl.ds(tgt * n_per, n_per)] = y_blk
                rdma = pltpu.make_async_remote_copy(
                    src_ref=y_ref.at[:, pl.ds(tgt * n_per, n_per)],
                    dst_ref=out_ref.at[pl.ds(my_pos * m_per, m_per), :],
                    send_sem=send_sems.at[d - 1],
                    recv_sem=recv_sems.at[d - 1],
                    device_id=(tgt,),
                    device_id_type=pl.DeviceIdType.MESH,
                )
                rdma.start()
                rdmas.append(rdma)
        for rdma in rdmas:
            rdma.wait()

    return pl.pallas_call(
        body,
        out_shape=jax.ShapeDtypeStruct((N_DEV * m_per, n_per), jnp.float32),
        in_specs=[
            pl.BlockSpec(memory_space=pltpu.VMEM),
            pl.BlockSpec(memory_space=pltpu.VMEM),
        ],
        out_specs=pl.BlockSpec(memory_space=pltpu.VMEM),
        scratch_shapes=[
            pltpu.VMEM((m_per, n), jnp.float32),
            pltpu.SemaphoreType.DMA((N_DEV - 1,)),
            pltpu.SemaphoreType.DMA((N_DEV - 1,)),
        ],
        compiler_params=pltpu.CompilerParams(collective_id=0),
    )(x, w_mat)


# device time: 24520 ns/iter; 1.5797x vs baseline; 1.5797x over previous
import jax
import jax.numpy as jnp
from jax import lax
from jax.experimental import pallas as pl
from jax.experimental.pallas import tpu as pltpu

N_DEV = 4
_GELU_C = 0.7978845608028654
_ORDER = (2, 1, 3, 0)


def _gelu(y):
    return 0.5 * y * (1.0 + jnp.tanh(_GELU_C * (y + 0.044715 * y * y * y)))


def kernel(x, w_mat):
    m_per, k = x.shape
    _, n = w_mat.shape
    n_per = n // N_DEV

    def _wcopy(w_hbm, wbuf, wsems, tgt, slot, n_per):
        return pltpu.make_async_copy(
            w_hbm.at[:, pl.ds(tgt * n_per, n_per)], wbuf.at[slot], wsems.at[slot]
        )

    def body(x_ref, w_hbm, out_ref, wbuf, snd, rcv, wsems, send_sems, recv_sems):
        my_pos = lax.axis_index("i")

        barrier_sem = pltpu.get_barrier_semaphore()
        for d in range(1, N_DEV):
            pl.semaphore_signal(
                barrier_sem,
                inc=1,
                device_id=((my_pos + d) % N_DEV,),
                device_id_type=pl.DeviceIdType.MESH,
            )
        pl.semaphore_wait(barrier_sem, N_DEV - 1)

        tgt0 = (my_pos + _ORDER[0]) % N_DEV
        _wcopy(w_hbm, wbuf, wsems, tgt0, 0, n_per).start()

        rdmas = {}
        for s in range(N_DEV):
            d = _ORDER[s]
            tgt = (my_pos + d) % N_DEV
            if s + 1 < N_DEV:
                tgt_nxt = (my_pos + _ORDER[s + 1]) % N_DEV
                _wcopy(w_hbm, wbuf, wsems, tgt_nxt, (s + 1) % 2, n_per).start()
            _wcopy(w_hbm, wbuf, wsems, tgt, s % 2, n_per).wait()

            y_blk = _gelu(
                jnp.dot(x_ref[:, :], wbuf[s % 2], preferred_element_type=jnp.float32)
            )
            if d == 0:
                out_ref[pl.ds(my_pos * m_per, m_per), :] = y_blk
            else:
                snd[d - 1] = y_blk.astype(jnp.bfloat16)
                rdma = pltpu.make_async_remote_copy(
                    src_ref=snd.at[d - 1],
                    dst_ref=rcv.at[d - 1],
                    send_sem=send_sems.at[d - 1],
                    recv_sem=recv_sems.at[d - 1],
                    device_id=(tgt,),
                    device_id_type=pl.DeviceIdType.MESH,
                )
                rdma.start()
                rdmas[d] = rdma

        for d in (1, 3, 2):
            rdmas[d].wait()
            src_pos = (my_pos - d) % N_DEV
            out_ref[pl.ds(src_pos * m_per, m_per), :] = rcv[d - 1].astype(
                jnp.float32
            )

    return pl.pallas_call(
        body,
        out_shape=jax.ShapeDtypeStruct((N_DEV * m_per, n_per), jnp.float32),
        in_specs=[
            pl.BlockSpec(memory_space=pltpu.VMEM),
            pl.BlockSpec(memory_space=pltpu.MemorySpace.HBM),
        ],
        out_specs=pl.BlockSpec(memory_space=pltpu.VMEM),
        scratch_shapes=[
            pltpu.VMEM((2, k, n_per), jnp.float32),
            pltpu.VMEM((N_DEV - 1, m_per, n_per), jnp.bfloat16),
            pltpu.VMEM((N_DEV - 1, m_per, n_per), jnp.bfloat16),
            pltpu.SemaphoreType.DMA((2,)),
            pltpu.SemaphoreType.DMA((N_DEV - 1,)),
            pltpu.SemaphoreType.DMA((N_DEV - 1,)),
        ],
        compiler_params=pltpu.CompilerParams(collective_id=0),
    )(x, w_mat)


# device time: 23789 ns/iter; 1.6283x vs baseline; 1.0307x over previous
import jax
import jax.numpy as jnp
from jax import lax
from jax.experimental import pallas as pl
from jax.experimental.pallas import tpu as pltpu

N_DEV = 4
_GELU_C = 0.7978845608028654
_SEQ = ((2, 0), (2, 1), (1, 0), (1, 1), (3, 0), (3, 1), (0, 0), (0, 1))
_DRAIN = ((1, 0), (1, 1), (3, 0), (3, 1), (2, 0), (2, 1))


def _gelu(y):
    return 0.5 * y * (1.0 + jnp.tanh(_GELU_C * (y + 0.044715 * y * y * y)))


def kernel(x, w_mat):
    m_per, k = x.shape
    _, n = w_mat.shape
    n_per = n // N_DEV
    n_sub = n_per // 2

    def body(
        x_hbm, w_hbm, out_hbm,
        x_vmem, wbuf, snd, rcv, stage,
        x_sem, w_sems, out_sems, send_sems, recv_sems,
    ):
        my_pos = lax.axis_index("i")

        def wcopy(d, h, slot):
            tgt = (my_pos + d) % N_DEV
            return pltpu.make_async_copy(
                w_hbm.at[:, pl.ds(tgt * n_per + h * n_sub, n_sub)],
                wbuf.at[slot],
                w_sems.at[slot],
            )

        xcopy = pltpu.make_async_copy(x_hbm, x_vmem, x_sem)
        xcopy.start()
        wcopy(*_SEQ[0], 0).start()

        barrier_sem = pltpu.get_barrier_semaphore()
        for d in range(1, N_DEV):
            pl.semaphore_signal(
                barrier_sem,
                inc=1,
                device_id=((my_pos + d) % N_DEV,),
                device_id_type=pl.DeviceIdType.MESH,
            )
        pl.semaphore_wait(barrier_sem, N_DEV - 1)
        xcopy.wait()

        out_dma = {0: None, 1: None}
        out_uses = [0]

        def stage_out(block_f32, row_pos, h):
            slot = out_uses[0] % 2
            out_uses[0] += 1
            if out_dma[slot] is not None:
                out_dma[slot].wait()
            stage[slot] = block_f32
            dma = pltpu.make_async_copy(
                stage.at[slot],
                out_hbm.at[pl.ds(row_pos * m_per, m_per), pl.ds(h * n_sub, n_sub)],
                out_sems.at[slot],
            )
            dma.start()
            out_dma[slot] = dma

        rdmas = {}
        for s, (d, h) in enumerate(_SEQ):
            if s + 1 < len(_SEQ):
                wcopy(*_SEQ[s + 1], (s + 1) % 2).start()
            wcopy(d, h, s % 2).wait()
            y_blk = _gelu(
                jnp.dot(x_vmem[:, :], wbuf[s % 2], preferred_element_type=jnp.float32)
            )
            if d == 0:
                stage_out(y_blk, my_pos, h)
            else:
                idx = (d - 1) * 2 + h
                snd[idx] = y_blk.astype(jnp.bfloat16)
                rdma = pltpu.make_async_remote_copy(
                    src_ref=snd.at[idx],
                    dst_ref=rcv.at[idx],
                    send_sem=send_sems.at[idx],
                    recv_sem=recv_sems.at[idx],
                    device_id=((my_pos + d) % N_DEV,),
                    device_id_type=pl.DeviceIdType.MESH,
                )
                rdma.start()
                rdmas[(d, h)] = rdma

        for d, h in _DRAIN:
            rdmas[(d, h)].wait()
            src_pos = (my_pos - d) % N_DEV
            stage_out(rcv[(d - 1) * 2 + h].astype(jnp.float32), src_pos, h)
        for slot in (0, 1):
            if out_dma[slot] is not None:
                out_dma[slot].wait()

    return pl.pallas_call(
        body,
        out_shape=jax.ShapeDtypeStruct((N_DEV * m_per, n_per), jnp.float32),
        in_specs=[
            pl.BlockSpec(memory_space=pltpu.MemorySpace.HBM),
            pl.BlockSpec(memory_space=pltpu.MemorySpace.HBM),
        ],
        out_specs=pl.BlockSpec(memory_space=pltpu.MemorySpace.HBM),
        scratch_shapes=[
            pltpu.VMEM((m_per, k), jnp.float32),
            pltpu.VMEM((2, k, n_sub), jnp.float32),
            pltpu.VMEM((6, m_per, n_sub), jnp.bfloat16),
            pltpu.VMEM((6, m_per, n_sub), jnp.bfloat16),
            pltpu.VMEM((2, m_per, n_sub), jnp.float32),
            pltpu.SemaphoreType.DMA,
            pltpu.SemaphoreType.DMA((2,)),
            pltpu.SemaphoreType.DMA((2,)),
            pltpu.SemaphoreType.DMA((6,)),
            pltpu.SemaphoreType.DMA((6,)),
        ],
        compiler_params=pltpu.CompilerParams(collective_id=0),
    )(x, w_mat)
